# baseline (speedup 1.0000x reference)
import functools

import jax
import jax.numpy as jnp
from jax import lax
from jax.experimental import pallas as pl
from jax.experimental.pallas import tpu as pltpu
from jax.experimental.pallas import tpu_sc as plsc

NBUF = 3
CHUNK = 32


@jax.jit
def _gather_rows_sc(idx_flat, pe):
    n = idx_flat.shape[0]
    d = pe.shape[1]
    info = plsc.get_sparse_core_info()
    num_cores = info.num_cores
    nw = num_cores * info.num_subcores
    n_per_w = n // nw
    n_ch = n_per_w // CHUNK
    assert n_per_w * nw == n and n_ch * CHUNK == n_per_w

    mesh = plsc.VectorSubcoreMesh(core_axis_name="c", subcore_axis_name="s")

    @functools.partial(
        pl.kernel,
        mesh=mesh,
        out_type=jax.ShapeDtypeStruct((n, d), jnp.float32),
        scratch_types=[
            pltpu.VMEM((n_per_w,), jnp.int32),
            pltpu.VMEM((NBUF, CHUNK, d), jnp.float32),
            pltpu.SemaphoreType.DMA,
            pltpu.SemaphoreType.DMA,
            pltpu.SemaphoreType.DMA,
        ],
    )
    def k(idx_hbm, pe_hbm, out_hbm, idx_v, rows_v, sem0, sem1, sem2):
        sems = (sem0, sem1, sem2)
        wid = lax.axis_index("s") * num_cores + lax.axis_index("c")
        base = wid * n_per_w
        pltpu.sync_copy(idx_hbm.at[pl.ds(base, n_per_w)], idx_v)

        def start_gather(c):
            b = c % NBUF
            return pltpu.async_copy(
                pe_hbm.at[idx_v.at[pl.ds(c * CHUNK, CHUNK)]],
                rows_v.at[b],
                sems[b],
            )

        def start_store(c):
            b = c % NBUF
            return pltpu.async_copy(
                rows_v.at[b],
                out_hbm.at[pl.ds(base + c * CHUNK, CHUNK)],
                sems[b],
            )

        gathers = {}
        stores = {}
        for c in range(min(NBUF - 1, n_ch)):
            gathers[c] = start_gather(c)
        for c in range(n_ch):
            gathers[c].wait()
            stores[c] = start_store(c)
            nxt = c + NBUF - 1
            if nxt < n_ch:
                if c >= 1:
                    stores[c - 1].wait()
                gathers[nxt] = start_gather(nxt)
        for c in range(max(0, n_ch - NBUF + 1), n_ch):
            stores[c].wait()

    return k(idx_flat, pe)


def kernel(time, pe):
    out = _gather_rows_sc(time.reshape(-1), pe)
    return out.reshape(time.shape + (pe.shape[1],))


# speedup vs baseline: 1.0357x; 1.0357x over previous
import functools

import jax
import jax.numpy as jnp
from jax import lax
from jax.experimental import pallas as pl
from jax.experimental.pallas import tpu as pltpu
from jax.experimental.pallas import tpu_sc as plsc

NBUF = 7
CHUNK = 16


@jax.jit
def _gather_rows_sc(idx_flat, pe):
    n = idx_flat.shape[0]
    d = pe.shape[1]
    info = plsc.get_sparse_core_info()
    num_cores = info.num_cores
    nw = num_cores * info.num_subcores
    n_per_w = n // nw
    n_ch = n_per_w // CHUNK
    assert n_per_w * nw == n and n_ch * CHUNK == n_per_w

    mesh = plsc.VectorSubcoreMesh(core_axis_name="c", subcore_axis_name="s")

    @functools.partial(
        pl.kernel,
        mesh=mesh,
        out_type=jax.ShapeDtypeStruct((n, d), jnp.float32),
        scratch_types=[
            pltpu.VMEM((n_per_w,), jnp.int32),
            pltpu.VMEM((NBUF, CHUNK, d), jnp.float32),
            pltpu.SemaphoreType.DMA((NBUF,)),
        ],
    )
    def k(idx_hbm, pe_hbm, out_hbm, idx_v, rows_v, sems):
        wid = lax.axis_index("s") * num_cores + lax.axis_index("c")
        base = wid * n_per_w
        pltpu.sync_copy(idx_hbm.at[pl.ds(base, n_per_w)], idx_v)

        def start_gather(c):
            b = c % NBUF
            return pltpu.async_copy(
                pe_hbm.at[idx_v.at[pl.ds(c * CHUNK, CHUNK)]],
                rows_v.at[b],
                sems.at[b],
            )

        def start_store(c):
            b = c % NBUF
            return pltpu.async_copy(
                rows_v.at[b],
                out_hbm.at[pl.ds(base + c * CHUNK, CHUNK)],
                sems.at[b],
            )

        gathers = {}
        stores = {}
        for c in range(min(NBUF - 1, n_ch)):
            gathers[c] = start_gather(c)
        for c in range(n_ch):
            gathers[c].wait()
            stores[c] = start_store(c)
            nxt = c + NBUF - 1
            if nxt < n_ch:
                if c >= 1:
                    stores[c - 1].wait()
                gathers[nxt] = start_gather(nxt)
        for c in range(max(0, n_ch - NBUF + 1), n_ch):
            stores[c].wait()

    return k(idx_flat, pe)


def kernel(time, pe):
    out = _gather_rows_sc(time.reshape(-1), pe)
    return out.reshape(time.shape + (pe.shape[1],))
